# baseline (speedup 1.0000x reference)
import functools

import jax
import jax.numpy as jnp
from jax import lax
from jax.experimental import pallas as pl
from jax.experimental.pallas import tpu as pltpu
from jax.experimental.pallas import tpu_sc as plsc

_B = 16384
_D = 64
_L = 16

_info = plsc.get_sparse_core_info()
_NC, _NS = _info.num_cores, _info.num_subcores
_NW = _NC * _NS
_BPW = _B // _NW

_mesh = plsc.VectorSubcoreMesh(core_axis_name="c", subcore_axis_name="s")


@functools.partial(
    pl.kernel,
    out_type=[
        jax.ShapeDtypeStruct((_B,), jnp.float32),
        jax.ShapeDtypeStruct((_NW, _L), jnp.float32),
    ],
    mesh=_mesh,
    compiler_params=pltpu.CompilerParams(use_tc_tiling_on_sc=False),
    scratch_types=[
        pltpu.VMEM((_BPW,), jnp.int32),
        pltpu.VMEM((_BPW,), jnp.int32),
        pltpu.VMEM((_BPW, _D), jnp.float32),
        pltpu.VMEM((_BPW, _D), jnp.float32),
        pltpu.VMEM((_BPW,), jnp.float32),
        pltpu.VMEM((_L,), jnp.float32),
        pltpu.SemaphoreType.DMA,
        pltpu.SemaphoreType.DMA,
    ],
)
def _mf_kernel(uidx_hbm, iidx_hbm, utab_hbm, itab_hbm, pred_hbm, partials_hbm,
               uidx_v, iidx_v, urows_v, irows_v, pred_v, accsq_v, sem_u, sem_i):
    wid = lax.axis_index("s") * _NC + lax.axis_index("c")
    base = wid * _BPW

    pltpu.sync_copy(uidx_hbm.at[pl.ds(base, _BPW)], uidx_v)
    pltpu.sync_copy(iidx_hbm.at[pl.ds(base, _BPW)], iidx_v)
    cu = pltpu.async_copy(utab_hbm.at[uidx_v], urows_v, sem_u)
    ci = pltpu.async_copy(itab_hbm.at[iidx_v], irows_v, sem_i)
    cu.wait()
    ci.wait()

    lane = lax.iota(jnp.int32, _L)

    def chunk_body(c, accsq):
        base_r = pl.multiple_of(c * _L, _L)
        preds = jnp.zeros((_L,), jnp.float32)
        for r in range(_L):
            prod = jnp.zeros((_L,), jnp.float32)
            for k in range(_D // _L):
                u = urows_v[base_r + r, pl.ds(k * _L, _L)]
                i = irows_v[base_r + r, pl.ds(k * _L, _L)]
                prod = prod + u * i
                accsq = accsq + (u * u + i * i)
            for sh in (8, 4, 2, 1):
                prod = prod + prod.at[lane ^ sh].get(mode="promise_in_bounds")
            preds = jnp.where(lane == r, prod, preds)
        pred_v[pl.ds(base_r, _L)] = preds
        return accsq

    accsq = lax.fori_loop(0, _BPW // _L, chunk_body,
                          jnp.zeros((_L,), jnp.float32))
    accsq_v[...] = accsq

    pltpu.sync_copy(pred_v, pred_hbm.at[pl.ds(base, _BPW)])
    pltpu.sync_copy(accsq_v, partials_hbm.at[wid])


def kernel(user_indices, item_indices, user_embedding_weight, item_embedding_weight):
    pred, partials = _mf_kernel(
        user_indices.astype(jnp.int32),
        item_indices.astype(jnp.int32),
        user_embedding_weight,
        item_embedding_weight,
    )
    reg_loss = 0.5 * jnp.sum(partials) / float(_B)
    return pred, reg_loss


# speedup vs baseline: 2.1772x; 2.1772x over previous
import functools

import jax
import jax.numpy as jnp
from jax import lax
from jax.experimental import pallas as pl
from jax.experimental.pallas import tpu as pltpu
from jax.experimental.pallas import tpu_sc as plsc

_B = 16384
_D = 64
_L = 16
_G = 8

_info = plsc.get_sparse_core_info()
_NC, _NS = _info.num_cores, _info.num_subcores
_NW = _NC * _NS
_BPW = _B // _NW
_CH = 16
_NCH = _BPW // _CH

_mesh = plsc.VectorSubcoreMesh(core_axis_name="c", subcore_axis_name="s")


@functools.partial(
    pl.kernel,
    out_type=[
        jax.ShapeDtypeStruct((_B,), jnp.float32),
        jax.ShapeDtypeStruct((_NW, _L), jnp.float32),
    ],
    mesh=_mesh,
    scratch_types=[
        pltpu.VMEM((_BPW,), jnp.int32),
        pltpu.VMEM((_BPW,), jnp.int32),
        pltpu.VMEM((2, _CH, _G, _D), jnp.float32),
        pltpu.VMEM((2, _CH, _G, _D), jnp.float32),
        pltpu.VMEM((_BPW,), jnp.float32),
        pltpu.VMEM((_L,), jnp.float32),
        pltpu.SemaphoreType.DMA,
        pltpu.SemaphoreType.DMA,
        pltpu.SemaphoreType.DMA,
        pltpu.SemaphoreType.DMA,
    ],
)
def _mf_kernel(uidx_hbm, iidx_hbm, utab_hbm, itab_hbm, pred_hbm, partials_hbm,
               uidx_v, iidx_v, ubuf, ibuf, pred_v, accsq_v,
               sem_u0, sem_u1, sem_i0, sem_i1):
    wid = lax.axis_index("s") * _NC + lax.axis_index("c")
    base = wid * _BPW

    pltpu.sync_copy(uidx_hbm.at[pl.ds(base, _BPW)], uidx_v)
    pltpu.sync_copy(iidx_hbm.at[pl.ds(base, _BPW)], iidx_v)

    sems_u = (sem_u0, sem_u1)
    sems_i = (sem_i0, sem_i1)
    lane = lax.iota(jnp.int32, _L)

    def issue(c, slot):
        c = jnp.minimum(c, _NCH - 1)
        off = pl.multiple_of(c * _CH, _CH)
        gu = uidx_v[pl.ds(off, _CH)] >> 3
        gi = iidx_v[pl.ds(off, _CH)] >> 3
        for j in range(_CH):
            pltpu.async_copy(utab_hbm.at[gu[j]], ubuf.at[slot, j], sems_u[slot])
            pltpu.async_copy(itab_hbm.at[gi[j]], ibuf.at[slot, j], sems_i[slot])

    def wait(slot):
        for j in range(_CH):
            pltpu.make_async_copy(utab_hbm.at[0], ubuf.at[slot, j],
                                  sems_u[slot]).wait()
            pltpu.make_async_copy(itab_hbm.at[0], ibuf.at[slot, j],
                                  sems_i[slot]).wait()

    def compute(c, slot, accsq):
        off = pl.multiple_of(c * _CH, _CH)
        su = uidx_v[pl.ds(off, _CH)] & 7
        si = iidx_v[pl.ds(off, _CH)] & 7
        preds = jnp.zeros((_L,), jnp.float32)
        ub = ubuf.at[slot]
        ib = ibuf.at[slot]
        for j in range(_CH):
            ru = su[j]
            ri = si[j]
            prod = jnp.zeros((_L,), jnp.float32)
            for k in range(_D // _L):
                u = ub[j, ru, pl.ds(k * _L, _L)]
                i = ib[j, ri, pl.ds(k * _L, _L)]
                prod = prod + u * i
                accsq = accsq + (u * u + i * i)
            for sh in (8, 4, 2, 1):
                prod = prod + prod.at[lane ^ sh].get(mode="promise_in_bounds")
            preds = jnp.where(lane == j, prod, preds)
        pred_v[pl.ds(pl.multiple_of(c * _CH, _CH), _L)] = preds
        return accsq

    issue(jnp.int32(0), 0)
    issue(jnp.int32(1), 1)

    def body(m, accsq):
        c0 = m * 2
        wait(0)
        accsq = compute(c0, 0, accsq)
        issue(c0 + 2, 0)
        wait(1)
        accsq = compute(c0 + 1, 1, accsq)
        issue(c0 + 3, 1)
        return accsq

    accsq = lax.fori_loop(0, _NCH // 2, body, jnp.zeros((_L,), jnp.float32))
    wait(0)
    wait(1)
    accsq_v[...] = accsq

    pltpu.sync_copy(pred_v, pred_hbm.at[pl.ds(base, _BPW)])
    pltpu.sync_copy(accsq_v, partials_hbm.at[wid])


def kernel(user_indices, item_indices, user_embedding_weight, item_embedding_weight):
    utab3 = user_embedding_weight.reshape(1000000 // _G, _G, _D)
    itab3 = item_embedding_weight.reshape(1000000 // _G, _G, _D)
    pred, partials = _mf_kernel(
        user_indices.astype(jnp.int32),
        item_indices.astype(jnp.int32),
        utab3,
        itab3,
    )
    reg_loss = 0.5 * jnp.sum(partials) / float(_B)
    return pred, reg_loss
